# Initial kernel scaffold; baseline (speedup 1.0000x reference)
import jax
import jax.numpy as jnp
from jax.experimental import pallas as pl


def kernel(cent_obs, actions, params, rows, cols):
    raise NotImplementedError("write your pallas kernel here")


# speedup vs baseline: 14.1999x; 14.1999x over previous
import jax
import jax.numpy as jnp
from jax import lax
from jax.experimental import pallas as pl
from jax.experimental.pallas import tpu as pltpu

N_AGENTS = 20
BATCH = 2500
INV_NF = 12
HID = 64
N_LAYERS = 2
DEG = float(N_AGENTS - 1)

B_TILE = 32
BATCH_PAD = 2560


def _silu(x):
    return x * jax.nn.sigmoid(x)


def _egnn_body(x_ref, loc_ref, act_ref, *refs):
    out_ref = refs[-1]
    w = [r[...] for r in refs[:-1]]
    B, A, H = B_TILE, N_AGENTS, HID
    BA = B * A

    k = iter(range(len(w)))
    Wemb, bemb = w[next(k)], w[next(k)]

    h = jnp.dot(x_ref[...], Wemb, preferred_element_type=jnp.float32) + bemb
    loc = loc_ref[...]
    v = act_ref[...]

    locr = loc.reshape(B, A, 2)
    cd0 = locr[:, :, None, :] - locr[:, None, :, :]
    ea = jnp.sum(cd0 * cd0, axis=-1, keepdims=True)

    ii = lax.broadcasted_iota(jnp.int32, (1, A, A, 1), 1)
    jj = lax.broadcasted_iota(jnp.int32, (1, A, A, 1), 2)
    offdiag = (ii != jj).astype(jnp.float32)

    for _ in range(N_LAYERS):
        (W1r, W1c, wr, we, b1, W2, b2, Wc1, bc1, wc2,
         Wn1h, Wn1a, bn1, Wn2, bn2, Wv1, bv1, wv2, bv2) = (
            w[next(k)] for _ in range(19))

        locr = loc.reshape(B, A, 2)
        cd = locr[:, :, None, :] - locr[:, None, :, :]
        radial = jnp.sum(cd * cd, axis=-1, keepdims=True)
        cdn = cd / (jnp.sqrt(radial) + 1.0)

        hr = jnp.dot(h, W1r, preferred_element_type=jnp.float32)
        hc = jnp.dot(h, W1c, preferred_element_type=jnp.float32)
        pre = (hr.reshape(B, A, 1, H) + hc.reshape(B, 1, A, H)
               + radial * wr + ea * we + b1)
        e1 = _silu(pre).reshape(BA * A, H)
        m = _silu(jnp.dot(e1, W2, preferred_element_type=jnp.float32) + b2)

        c1 = _silu(jnp.dot(m, Wc1, preferred_element_type=jnp.float32) + bc1)
        s = jnp.sum(c1.reshape(B, A, A, H) * wc2, axis=-1, keepdims=True)
        agg = jnp.sum(cdn * s, axis=2) / DEG

        velf = (jnp.sum(_silu(jnp.dot(h, Wv1, preferred_element_type=jnp.float32)
                              + bv1) * wv2, axis=-1, keepdims=True) + bv2)
        v = velf * v + agg.reshape(BA, 2)
        loc = loc + v

        nag = jnp.sum(m.reshape(B, A, A, H) * offdiag, axis=2).reshape(BA, H)
        n1 = _silu(jnp.dot(h, Wn1h, preferred_element_type=jnp.float32)
                   + jnp.dot(nag, Wn1a, preferred_element_type=jnp.float32) + bn1)
        h = h + jnp.dot(n1, Wn2, preferred_element_type=jnp.float32) + bn2

    wq, bq = w[next(k)], w[next(k)]
    q = jnp.sum(jnp.tanh(h) * wq, axis=-1, keepdims=True) + bq
    out_ref[...] = jnp.sum(q.reshape(B, A, 1), axis=1) / float(A)


def kernel(cent_obs, actions, params, rows, cols):
    del rows, cols
    N = BATCH * N_AGENTS
    pad_n = (BATCH_PAD - BATCH) * N_AGENTS

    x = cent_obs.reshape(N, INV_NF + 4)
    loc0 = x[:, INV_NF:INV_NF + 2]
    xp = jnp.pad(x, ((0, pad_n), (0, 0)))
    locp = jnp.pad(loc0, ((0, pad_n), (0, 0)))
    actp = jnp.pad(actions, ((0, pad_n), (0, 0)))

    Wemb, bemb = params["emb"]
    wlist = [jnp.pad(Wemb, ((0, 4), (0, 0))), bemb.reshape(1, HID)]
    for layer in params["layers"]:
        W1, b1 = layer["edge1"]
        W2, b2 = layer["edge2"]
        Wn1, bn1 = layer["node1"]
        Wn2, bn2 = layer["node2"]
        Wc1, bc1 = layer["coord1"]
        (Wc2,) = layer["coord2"]
        Wv1, bv1 = layer["vel1"]
        Wv2, bv2 = layer["vel2"]
        wlist += [
            W1[:HID], W1[HID:2 * HID], W1[2 * HID:2 * HID + 1],
            W1[2 * HID + 1:], b1.reshape(1, HID),
            W2, b2.reshape(1, HID),
            Wc1, bc1.reshape(1, HID), Wc2.reshape(1, HID),
            Wn1[:HID], Wn1[HID:], bn1.reshape(1, HID),
            Wn2, bn2.reshape(1, HID),
            Wv1, bv1.reshape(1, HID), Wv2.reshape(1, HID), bv2.reshape(1, 1),
        ]
    Wq, bq = params["critic"]
    wlist += [Wq.reshape(1, HID), bq.reshape(1, 1)]

    grid = (BATCH_PAD // B_TILE,)
    row_spec = lambda width: pl.BlockSpec((B_TILE * N_AGENTS, width),
                                          lambda i: (i, 0))
    w_specs = [pl.BlockSpec(wl.shape, lambda i: (0, 0)) for wl in wlist]

    out = pl.pallas_call(
        _egnn_body,
        grid=grid,
        in_specs=[row_spec(INV_NF + 4), row_spec(2), row_spec(2)] + w_specs,
        out_specs=pl.BlockSpec((B_TILE, 1), lambda i: (i, 0)),
        out_shape=jax.ShapeDtypeStruct((BATCH_PAD, 1), jnp.float32),
        compiler_params=pltpu.CompilerParams(
            dimension_semantics=("parallel",)),
    )(xp, locp, actp, *wlist)
    return out[:BATCH]
